# Initial kernel scaffold; baseline (speedup 1.0000x reference)
import jax
import jax.numpy as jnp
from jax.experimental import pallas as pl


def kernel(obj_vecs, predi_vecs, edges, W1a, b1a, W1b, b1b, W2a, b2a, W2b, b2b):
    raise NotImplementedError("write your pallas kernel here")


# speedup vs baseline: 3.4051x; 3.4051x over previous
import functools

import jax
import jax.numpy as jnp
from jax import lax
from jax.experimental import pallas as pl
from jax.experimental.pallas import tpu as pltpu
from jax.experimental.pallas import tpu_sc as plsc

O = 10000
T = 320000
D = 128

NC = 2
NS = 16
NW = NC * NS
L = 16

EPW = T // NW
B = 80
NCHUNK = EPW // B
OP = 10240
RPT = OP // NS

_mesh = plsc.VectorSubcoreMesh(core_axis_name="c", subcore_axis_name="s")


@functools.partial(
    pl.kernel,
    out_type=jax.ShapeDtypeStruct((T, D), jnp.float32),
    mesh=_mesh,
    scratch_types=[
        pltpu.VMEM((B,), jnp.int32),
        pltpu.VMEM((B,), jnp.int32),
        pltpu.VMEM((B, D), jnp.float32),
        pltpu.VMEM((B, D), jnp.float32),
        pltpu.SemaphoreType.DMA,
        pltpu.SemaphoreType.DMA,
    ],
)
def _gather_add(ps_hbm, po_hbm, sidx_hbm, oidx_hbm, out_hbm,
                idx_s, idx_o, rows_a, rows_b, sem_a, sem_b):
    wid = lax.axis_index("s") * NC + lax.axis_index("c")
    base = wid * EPW

    def chunk(j, carry):
        off = base + j * B
        pltpu.sync_copy(sidx_hbm.at[pl.ds(off, B)], idx_s)
        pltpu.sync_copy(oidx_hbm.at[pl.ds(off, B)], idx_o)
        cp_a = pltpu.async_copy(ps_hbm.at[idx_s], rows_a, sem_a)
        cp_b = pltpu.async_copy(po_hbm.at[idx_o], rows_b, sem_b)
        cp_a.wait()
        cp_b.wait()

        def row(r, c2):
            for col in range(D // L):
                sl = pl.ds(col * L, L)
                rows_a[r, sl] = rows_a[r, sl] + rows_b[r, sl]
            return c2

        lax.fori_loop(0, B, row, 0)
        pltpu.sync_copy(rows_a, out_hbm.at[pl.ds(off, B)])
        return carry

    lax.fori_loop(0, NCHUNK, chunk, 0)


@functools.partial(
    pl.kernel,
    out_type=(jax.ShapeDtypeStruct((NC, OP, D), jnp.float32),
              jax.ShapeDtypeStruct((NC * OP,), jnp.float32)),
    mesh=_mesh,
    scratch_types=[
        pltpu.VMEM((B,), jnp.int32),
        pltpu.VMEM((B,), jnp.int32),
        pltpu.VMEM((B, D), jnp.float32),
        pltpu.VMEM((B, D), jnp.float32),
        pltpu.VMEM((B,), jnp.float32),
        pltpu.VMEM_SHARED((OP, D), jnp.float32),
        pltpu.VMEM_SHARED((OP,), jnp.float32),
    ],
)
def _scatter_pool(sval_hbm, oval_hbm, sidx_hbm, oidx_hbm,
                  pooled_hbm, cnt_hbm,
                  idx_s, idx_o, val_s, val_o, cnt_stage,
                  pooled_sp, cnt_sp):
    c = lax.axis_index("c")
    s = lax.axis_index("s")
    base = (c * NS + s) * EPW

    zero16 = jnp.zeros((L,), jnp.float32)
    ones16 = jnp.ones((L,), jnp.float32)

    def fill_z(r, carry):
        for col in range(D // L):
            val_s[r, pl.ds(col * L, L)] = zero16
        return carry

    lax.fori_loop(0, B, fill_z, 0)
    for q in range(B // L):
        cnt_stage[pl.ds(q * L, L)] = zero16

    for k in range(RPT // B):
        row0 = s * RPT + k * B
        pltpu.sync_copy(val_s, pooled_sp.at[pl.ds(row0, B)])
        pltpu.sync_copy(cnt_stage, cnt_sp.at[pl.ds(row0, B)])

    for q in range(B // L):
        cnt_stage[pl.ds(q * L, L)] = ones16
    plsc.subcore_barrier()

    def chunk(j, carry):
        off = base + j * B
        pltpu.sync_copy(sidx_hbm.at[pl.ds(off, B)], idx_s)
        pltpu.sync_copy(oidx_hbm.at[pl.ds(off, B)], idx_o)
        pltpu.sync_copy(sval_hbm.at[pl.ds(off, B)], val_s)
        pltpu.sync_copy(oval_hbm.at[pl.ds(off, B)], val_o)
        pltpu.sync_copy(val_s, pooled_sp.at[idx_s], add=True)
        pltpu.sync_copy(val_o, pooled_sp.at[idx_o], add=True)
        pltpu.sync_copy(cnt_stage, cnt_sp.at[idx_s], add=True)
        pltpu.sync_copy(cnt_stage, cnt_sp.at[idx_o], add=True)
        return carry

    lax.fori_loop(0, NCHUNK, chunk, 0)
    plsc.subcore_barrier()

    for k in range(RPT // B):
        row0 = s * RPT + k * B
        pltpu.sync_copy(pooled_sp.at[pl.ds(row0, B)], val_s)
        pltpu.sync_copy(val_s, pooled_hbm.at[c, pl.ds(row0, B)])
        pltpu.sync_copy(cnt_sp.at[pl.ds(row0, B)], cnt_stage)
        pltpu.sync_copy(cnt_stage, cnt_hbm.at[pl.ds(c * OP + row0, B)])


def _proj_body(obj_ref, ws_ref, wo_ref, ps_ref, po_ref):
    x = obj_ref[...]
    ps_ref[...] = jnp.dot(x, ws_ref[...], preferred_element_type=jnp.float32)
    po_ref[...] = jnp.dot(x, wo_ref[...], preferred_element_type=jnp.float32)


def _edge_mlp_body(e_ref, p_ref, w1ap_ref, b1a_ref, w1b_ref, b1b_ref,
                   s_ref, pout_ref, o_ref):
    x = (e_ref[...]
         + jnp.dot(p_ref[...], w1ap_ref[...],
                   preferred_element_type=jnp.float32)
         + b1a_ref[...])
    h1 = jnp.maximum(x, 0.0)
    u = jnp.dot(h1, w1b_ref[...], preferred_element_type=jnp.float32)
    u = jnp.maximum(u + b1b_ref[...], 0.0)
    s_ref[...] = u[:, :D]
    pout_ref[...] = u[:, D:2 * D]
    o_ref[...] = u[:, 2 * D:]


def _node_mlp_body(pp_ref, cnt_ref, w2a_ref, b2a_ref, w2b_ref, b2b_ref,
                   out_ref):
    pooled = pp_ref[0] + pp_ref[1]
    counts = cnt_ref[0] + cnt_ref[1]
    counts = jnp.clip(counts, 1.0, 1000.0)
    pooled = pooled / counts
    h2 = jnp.maximum(
        jnp.dot(pooled, w2a_ref[...], preferred_element_type=jnp.float32)
        + b2a_ref[...], 0.0)
    out_ref[...] = jnp.maximum(
        jnp.dot(h2, w2b_ref[...], preferred_element_type=jnp.float32)
        + b2b_ref[...], 0.0)


_EDGE_BLK = 2000
_NODE_BLK = 2000


def kernel(obj_vecs, predi_vecs, edges, W1a, b1a, W1b, b1b, W2a, b2a,
           W2b, b2b):
    s_idx = edges[:, 0].astype(jnp.int32)
    o_idx = edges[:, 1].astype(jnp.int32)

    W1a_s = W1a[:D]
    W1a_p = W1a[D:2 * D]
    W1a_o = W1a[2 * D:]

    ps, po = pl.pallas_call(
        _proj_body,
        out_shape=(jax.ShapeDtypeStruct((O, D), jnp.float32),
                   jax.ShapeDtypeStruct((O, D), jnp.float32)),
    )(obj_vecs, W1a_s, W1a_o)

    ebase = _gather_add(ps, po, s_idx, o_idx)

    grid = (T // _EDGE_BLK,)
    blk = pl.BlockSpec((_EDGE_BLK, D), lambda i: (i, 0))
    full = lambda a, b: pl.BlockSpec((a, b), lambda i: (0, 0))
    sval, new_p, oval = pl.pallas_call(
        _edge_mlp_body,
        grid=grid,
        in_specs=[blk, blk, full(D, D), full(1, D), full(D, 3 * D),
                  full(1, 3 * D)],
        out_specs=(blk, blk, blk),
        out_shape=(jax.ShapeDtypeStruct((T, D), jnp.float32),
                   jax.ShapeDtypeStruct((T, D), jnp.float32),
                   jax.ShapeDtypeStruct((T, D), jnp.float32)),
    )(ebase, predi_vecs, W1a_p, b1a.reshape(1, D), W1b,
      b1b.reshape(1, 3 * D))

    pooled_parts, cnt_flat = _scatter_pool(sval, oval, s_idx, o_idx)
    cnt_parts = cnt_flat.reshape(NC, OP, 1)

    ngrid = (O // _NODE_BLK,)
    new_obj = pl.pallas_call(
        _node_mlp_body,
        grid=ngrid,
        in_specs=[pl.BlockSpec((NC, _NODE_BLK, D), lambda i: (0, i, 0)),
                  pl.BlockSpec((NC, _NODE_BLK, 1), lambda i: (0, i, 0)),
                  full(D, D), full(1, D), full(D, D), full(1, D)],
        out_specs=pl.BlockSpec((_NODE_BLK, D), lambda i: (i, 0)),
        out_shape=jax.ShapeDtypeStruct((O, D), jnp.float32),
    )(pooled_parts, cnt_parts, W2a, b2a.reshape(1, D), W2b,
      b2b.reshape(1, D))

    return (new_obj, new_p)
